# baseline (speedup 1.0000x reference)
import functools

import jax
import jax.numpy as jnp
from jax import lax
from jax.experimental import pallas as pl
from jax.experimental.pallas import tpu as pltpu
from jax.experimental.pallas import tpu_sc as plsc

NC = 2
NS = 16
NW = NC * NS
CHUNK = 128

F32 = jnp.float32


def _sc_mesh():
    return plsc.VectorSubcoreMesh(core_axis_name="c", subcore_axis_name="s",
                                  num_cores=NC, num_subcores=NS)


def _sc_segsum(feat, src4, dst4, n_rows):
    k = src4.shape[1] - 4
    assert k % 4 == 0 and n_rows % (NS * 8) == 0
    rpt = n_rows // NS
    zeros128 = jnp.zeros((n_rows, 128), F32)

    def body(feat_h, s_h, d_h, z128_h, out_h, sidx, didx, rows, acc, isem,
             gsem):
        cid = lax.axis_index("c")
        sid = lax.axis_index("s")
        wid = sid * NC + cid
        r0 = sid * rpt

        pltpu.sync_copy(z128_h.at[pl.ds(r0, rpt)], acc.at[pl.ds(r0, rpt)])
        plsc.subcore_barrier()

        def idx_load(j, p4):
            pltpu.async_copy(s_h.at[wid, j], sidx[p4], isem[p4])
            pltpu.async_copy(d_h.at[wid, j], didx[p4], isem[p4])

        def idx_wait(j, p4):
            pltpu.make_async_copy(s_h.at[wid, j], sidx[p4], isem[p4]).wait()
            pltpu.make_async_copy(d_h.at[wid, j], didx[p4], isem[p4]).wait()

        def gather(p4, p2):
            pltpu.async_copy(feat_h.at[sidx[p4]], rows[p2], gsem[p2])

        def gather_wait(p4, p2):
            pltpu.make_async_copy(feat_h.at[sidx[p4]], rows[p2],
                                  gsem[p2]).wait()

        def scat(p4, p2):
            pltpu.sync_copy(rows[p2], acc.at[didx[p4]], add=True)

        for p in range(4):
            idx_load(p, p)
        idx_wait(0, 0)
        gather(0, 0)
        idx_wait(1, 1)
        gather(1, 1)

        def slot(j, p4, p2):
            gather_wait(p4, p2)
            scat(p4, p2)
            idx_wait(j + 2, (p4 + 2) % 4)
            gather((p4 + 2) % 4, p2)
            idx_load(j + 4, p4)

        def step(t, carry):
            j0 = 4 * t
            slot(j0, 0, 0)
            slot(j0 + 1, 1, 1)
            slot(j0 + 2, 2, 0)
            slot(j0 + 3, 3, 1)
            return carry

        lax.fori_loop(0, k // 4, step, 0)
        gather_wait(0, 0)
        gather_wait(1, 1)
        idx_wait(k + 2, 2)
        idx_wait(k + 3, 3)
        plsc.subcore_barrier()

        pltpu.sync_copy(acc.at[pl.ds(r0, rpt)], out_h.at[cid, pl.ds(r0, rpt)])

    kern = pl.kernel(
        body,
        out_type=[jax.ShapeDtypeStruct((NC, n_rows, 128), F32)],
        mesh=_sc_mesh(),
        scratch_types=[
            [pltpu.VMEM((CHUNK,), jnp.int32) for _ in range(4)],
            [pltpu.VMEM((CHUNK,), jnp.int32) for _ in range(4)],
            [pltpu.VMEM((CHUNK, 128), F32) for _ in range(2)],
            pltpu.VMEM_SHARED((n_rows, 128), F32),
            [pltpu.SemaphoreType.DMA for _ in range(4)],
            [pltpu.SemaphoreType.DMA for _ in range(2)],
        ],
    )
    return kern(feat, src4, dst4, zeros128)[0]


def _sc_edge_segsum(ea4, dst4, n_rows):
    k = dst4.shape[1] - 4
    assert k % 4 == 0
    rpt = n_rows // NS
    zeros128 = jnp.zeros((n_rows, 128), F32)

    def body(ea_h, d_h, z128_h, out_h, didx, eat, acc, isem, lsem):
        cid = lax.axis_index("c")
        sid = lax.axis_index("s")
        wid = sid * NC + cid
        r0 = sid * rpt
        pltpu.sync_copy(z128_h.at[pl.ds(r0, rpt)], acc.at[pl.ds(r0, rpt)])
        plsc.subcore_barrier()

        def idx_load(j, p4):
            pltpu.async_copy(d_h.at[wid, j], didx[p4], isem[p4])

        def idx_wait(j, p4):
            pltpu.make_async_copy(d_h.at[wid, j], didx[p4], isem[p4]).wait()

        def load(j, p2):
            pltpu.async_copy(ea_h.at[wid, j], eat[p2], lsem[p2])

        def load_wait(j, p2):
            pltpu.make_async_copy(ea_h.at[wid, j], eat[p2], lsem[p2]).wait()

        def scat(p4, p2):
            pltpu.sync_copy(eat[p2], acc.at[didx[p4]], add=True)

        for p in range(4):
            idx_load(p, p)
        load(0, 0)
        load(1, 1)

        def slot(j, p4, p2):
            load_wait(j, p2)
            idx_wait(j, p4)
            scat(p4, p2)
            load(j + 2, p2)
            idx_load(j + 4, p4)

        def step(t, carry):
            j0 = 4 * t
            slot(j0, 0, 0)
            slot(j0 + 1, 1, 1)
            slot(j0 + 2, 2, 0)
            slot(j0 + 3, 3, 1)
            return carry

        lax.fori_loop(0, k // 4, step, 0)
        load_wait(k, 0)
        load_wait(k + 1, 1)
        idx_wait(k, 0)
        idx_wait(k + 1, 1)
        idx_wait(k + 2, 2)
        idx_wait(k + 3, 3)
        plsc.subcore_barrier()
        pltpu.sync_copy(acc.at[pl.ds(r0, rpt)], out_h.at[cid, pl.ds(r0, rpt)])

    kern = pl.kernel(
        body,
        out_type=[jax.ShapeDtypeStruct((NC, n_rows, 128), F32)],
        mesh=_sc_mesh(),
        scratch_types=[
            [pltpu.VMEM((CHUNK,), jnp.int32) for _ in range(4)],
            [pltpu.VMEM((CHUNK, 128), F32) for _ in range(2)],
            pltpu.VMEM_SHARED((n_rows, 128), F32),
            [pltpu.SemaphoreType.DMA for _ in range(4)],
            [pltpu.SemaphoreType.DMA for _ in range(2)],
        ],
    )
    return kern(ea4, dst4, zeros128)[0]


TC_BLK = 1000


def _tc_layer1(sa, ea, x, wl, we, wr, b, n):
    def body(sa_ref, ea_ref, x_ref, wl_ref, we_ref, wr_ref, b_ref, out_ref):
        s = sa_ref[0] + sa_ref[1]
        e = ea_ref[0] + ea_ref[1]
        inv = 1.0 / jnp.maximum(e[:, 16:17], 1.0)
        h = (
            jnp.dot(s * inv, wl_ref[...], preferred_element_type=F32,
                    precision=lax.Precision.HIGHEST)
            + jnp.dot(e[:, :16] * inv, we_ref[...], preferred_element_type=F32,
                      precision=lax.Precision.HIGHEST)
            + jnp.dot(x_ref[...], wr_ref[...], preferred_element_type=F32,
                      precision=lax.Precision.HIGHEST)
            + b_ref[...]
        )
        out_ref[...] = jnp.maximum(h, 0.0)

    full = lambda *shape: pl.BlockSpec(shape, lambda i: (0,) * len(shape))
    return pl.pallas_call(
        body,
        grid=(n // TC_BLK,),
        in_specs=[
            pl.BlockSpec((2, TC_BLK, 128), lambda i: (0, i, 0)),
            pl.BlockSpec((2, TC_BLK, 128), lambda i: (0, i, 0)),
            pl.BlockSpec((TC_BLK, 128), lambda i: (i, 0)),
            full(128, 128), full(16, 128), full(128, 128), full(128),
        ],
        out_specs=pl.BlockSpec((TC_BLK, 128), lambda i: (i, 0)),
        out_shape=jax.ShapeDtypeStruct((n, 128), F32),
    )(sa, ea, x, wl, we, wr, b)


def _tc_layer2_attn(sb, ea, h1, wl, we, wr, b, w1, b1, w2, n):
    def body(sb_ref, ea_ref, h1_ref, wl_ref, we_ref, wr_ref, b_ref,
             w1_ref, b1_ref, w2_ref, out_ref):
        s = sb_ref[0] + sb_ref[1]
        e = ea_ref[0] + ea_ref[1]
        inv = 1.0 / jnp.maximum(e[:, 16:17], 1.0)
        h2 = (
            jnp.dot(s * inv, wl_ref[...], preferred_element_type=F32,
                    precision=lax.Precision.HIGHEST)
            + jnp.dot(e[:, :16] * inv, we_ref[...], preferred_element_type=F32,
                      precision=lax.Precision.HIGHEST)
            + jnp.dot(h1_ref[...], wr_ref[...], preferred_element_type=F32,
                      precision=lax.Precision.HIGHEST)
            + b_ref[...]
        )
        h2a = h2[:, :64]
        h2b = h2[:, 64:]
        ua = jnp.tanh(jnp.dot(h2a, w1_ref[...], preferred_element_type=F32,
                              precision=lax.Precision.HIGHEST) + b1_ref[...])
        ub = jnp.tanh(jnp.dot(h2b, w1_ref[...], preferred_element_type=F32,
                              precision=lax.Precision.HIGHEST) + b1_ref[...])
        sa_s = jnp.dot(ua, w2_ref[...], preferred_element_type=F32,
                       precision=lax.Precision.HIGHEST)
        sb_s = jnp.dot(ub, w2_ref[...], preferred_element_type=F32,
                       precision=lax.Precision.HIGHEST)
        m = jnp.maximum(sa_s, sb_s)
        ea_w = jnp.exp(sa_s - m)
        eb_w = jnp.exp(sb_s - m)
        out_ref[...] = (ea_w * h2a + eb_w * h2b) / (ea_w + eb_w)

    full = lambda *shape: pl.BlockSpec(shape, lambda i: (0,) * len(shape))
    return pl.pallas_call(
        body,
        grid=(n // TC_BLK,),
        in_specs=[
            pl.BlockSpec((2, TC_BLK, 128), lambda i: (0, i, 0)),
            pl.BlockSpec((2, TC_BLK, 128), lambda i: (0, i, 0)),
            pl.BlockSpec((TC_BLK, 128), lambda i: (i, 0)),
            full(128, 128), full(16, 128), full(128, 128), full(128),
            full(64, 64), full(64), full(64, 1),
        ],
        out_specs=pl.BlockSpec((TC_BLK, 64), lambda i: (i, 0)),
        out_shape=jax.ShapeDtypeStruct((n, 64), F32),
    )(sb, ea, h1, wl, we, wr, b, w1, b1, w2)


def _block_diag(a, b):
    da0, da1 = a.shape
    db0, db1 = b.shape
    out = jnp.zeros((da0 + db0, da1 + db1), F32)
    out = out.at[:da0, :da1].set(a)
    out = out.at[da0:, da1:].set(b)
    return out


def kernel(x, edge_index, edge_attr, params):
    n, in_ch = x.shape
    e = edge_index.shape[1]
    hid = 64
    ech = edge_attr.shape[1]

    quantum = 4 * NW * CHUNK
    ep = ((e + quantum - 1) // quantum) * quantum
    n_rows = ((n + 1 + NS * 8 - 1) // (NS * 8)) * (NS * 8)

    k = ep // (NW * CHUNK)
    src4 = jnp.concatenate([
        jnp.zeros((ep,), jnp.int32).at[:e].set(edge_index[0])
        .reshape(NW, k, CHUNK),
        jnp.zeros((NW, 4, CHUNK), jnp.int32),
    ], axis=1)
    dst4 = jnp.concatenate([
        jnp.full((ep,), n, jnp.int32).at[:e].set(edge_index[1])
        .reshape(NW, k, CHUNK),
        jnp.full((NW, 4, CHUNK), n, jnp.int32),
    ], axis=1)
    ea128 = jnp.zeros((ep, 128), F32)
    ea128 = ea128.at[:e, :ech].set(edge_attr)
    ea128 = ea128.at[:e, 16].set(1.0)
    ea4 = jnp.concatenate([
        ea128.reshape(NW, k, CHUNK, 128),
        jnp.zeros((NW, 4, CHUNK, 128), F32),
    ], axis=1)

    c0, c1 = params["convs"][0], params["convs"][1]
    wl1 = _block_diag(c0[0]["Wl"][:64], c1[0]["Wl"][:64])
    we1 = jnp.concatenate([c0[0]["Wl"][64:], c1[0]["Wl"][64:]], axis=1)
    wr1 = _block_diag(c0[0]["Wr"], c1[0]["Wr"])
    b1v = jnp.concatenate([c0[0]["bl"] + c0[0]["br"],
                           c1[0]["bl"] + c1[0]["br"]])
    wl2 = _block_diag(c0[1]["Wl"][:64], c1[1]["Wl"][:64])
    we2 = jnp.concatenate([c0[1]["Wl"][64:], c1[1]["Wl"][64:]], axis=1)
    wr2 = _block_diag(c0[1]["Wr"], c1[1]["Wr"])
    b2v = jnp.concatenate([c0[1]["bl"] + c0[1]["br"],
                           c1[1]["bl"] + c1[1]["br"]])
    attn = params["attn"]

    eagg = _sc_edge_segsum(ea4, dst4, n_rows)
    sa = _sc_segsum(x, src4, dst4, n_rows)

    h1 = _tc_layer1(sa, eagg, x, wl1, we1, wr1, b1v, n)

    sb = _sc_segsum(h1, src4, dst4, n_rows)

    return _tc_layer2_attn(sb, eagg, h1, wl2, we2, wr2, b2v,
                           attn["W1"], attn["b1"], attn["W2"], n)


# speedup vs baseline: 1.1518x; 1.1518x over previous
import functools

import jax
import jax.numpy as jnp
from jax import lax
from jax.experimental import pallas as pl
from jax.experimental.pallas import tpu as pltpu
from jax.experimental.pallas import tpu_sc as plsc

NC = 2
NS = 16
NW = NC * NS
CHUNK = 128

F32 = jnp.float32


def _sc_mesh():
    return plsc.VectorSubcoreMesh(core_axis_name="c", subcore_axis_name="s",
                                  num_cores=NC, num_subcores=NS)


def _sc_segsum(feat, src4, dst4, n_rows):
    k = src4.shape[1] - 4
    assert k % 4 == 0 and n_rows % (NS * 8) == 0
    rpt = n_rows // NS
    zeros128 = jnp.zeros((n_rows, 128), F32)

    def body(feat_h, s_h, d_h, z128_h, out_h, sidx, didx, rows, acc, isem,
             gsem):
        cid = lax.axis_index("c")
        sid = lax.axis_index("s")
        wid = sid * NC + cid
        r0 = sid * rpt

        pltpu.sync_copy(z128_h.at[pl.ds(r0, rpt)], acc.at[pl.ds(r0, rpt)])
        plsc.subcore_barrier()

        def idx_load(j, p4):
            pltpu.async_copy(s_h.at[wid, j], sidx[p4], isem[p4])
            pltpu.async_copy(d_h.at[wid, j], didx[p4], isem[p4])

        def idx_wait(j, p4):
            pltpu.make_async_copy(s_h.at[wid, j], sidx[p4], isem[p4]).wait()
            pltpu.make_async_copy(d_h.at[wid, j], didx[p4], isem[p4]).wait()

        def gather(p4, p2):
            pltpu.async_copy(feat_h.at[sidx[p4]], rows[p2], gsem[p2])

        def gather_wait(p4, p2):
            pltpu.make_async_copy(feat_h.at[sidx[p4]], rows[p2],
                                  gsem[p2]).wait()

        def scat(p4, p2):
            pltpu.sync_copy(rows[p2], acc.at[didx[p4]], add=True)

        for p in range(4):
            idx_load(p, p)
        idx_wait(0, 0)
        gather(0, 0)

        def slot(j, p4, p2):
            idx_wait(j + 1, (p4 + 1) % 4)
            gather_wait(p4, p2)
            gather((p4 + 1) % 4, 1 - p2)
            scat(p4, p2)
            idx_load(j + 4, p4)

        def step(t, carry):
            j0 = 4 * t
            slot(j0, 0, 0)
            slot(j0 + 1, 1, 1)
            slot(j0 + 2, 2, 0)
            slot(j0 + 3, 3, 1)
            return carry

        lax.fori_loop(0, k // 4, step, 0)
        gather_wait(0, 0)
        idx_wait(k + 1, 1)
        idx_wait(k + 2, 2)
        idx_wait(k + 3, 3)
        plsc.subcore_barrier()

        pltpu.sync_copy(acc.at[pl.ds(r0, rpt)], out_h.at[cid, pl.ds(r0, rpt)])

    kern = pl.kernel(
        body,
        out_type=[jax.ShapeDtypeStruct((NC, n_rows, 128), F32)],
        mesh=_sc_mesh(),
        scratch_types=[
            [pltpu.VMEM((CHUNK,), jnp.int32) for _ in range(4)],
            [pltpu.VMEM((CHUNK,), jnp.int32) for _ in range(4)],
            [pltpu.VMEM((CHUNK, 128), F32) for _ in range(2)],
            pltpu.VMEM_SHARED((n_rows, 128), F32),
            [pltpu.SemaphoreType.DMA for _ in range(4)],
            [pltpu.SemaphoreType.DMA for _ in range(2)],
        ],
    )
    return kern(feat, src4, dst4, zeros128)[0]


def _sc_edge_segsum(ea4, dst4, n_rows):
    k = dst4.shape[1] - 4
    assert k % 4 == 0
    rpt = n_rows // NS
    zeros128 = jnp.zeros((n_rows, 128), F32)

    def body(ea_h, d_h, z128_h, out_h, didx, eat, acc, isem, lsem):
        cid = lax.axis_index("c")
        sid = lax.axis_index("s")
        wid = sid * NC + cid
        r0 = sid * rpt
        pltpu.sync_copy(z128_h.at[pl.ds(r0, rpt)], acc.at[pl.ds(r0, rpt)])
        plsc.subcore_barrier()

        def idx_load(j, p4):
            pltpu.async_copy(d_h.at[wid, j], didx[p4], isem[p4])

        def idx_wait(j, p4):
            pltpu.make_async_copy(d_h.at[wid, j], didx[p4], isem[p4]).wait()

        def load(j, p2):
            pltpu.async_copy(ea_h.at[wid, j], eat[p2], lsem[p2])

        def load_wait(j, p2):
            pltpu.make_async_copy(ea_h.at[wid, j], eat[p2], lsem[p2]).wait()

        def scat(p4, p2):
            pltpu.sync_copy(eat[p2], acc.at[didx[p4]], add=True)

        for p in range(4):
            idx_load(p, p)
        load(0, 0)
        load(1, 1)

        def slot(j, p4, p2):
            load_wait(j, p2)
            idx_wait(j, p4)
            scat(p4, p2)
            load(j + 2, p2)
            idx_load(j + 4, p4)

        def step(t, carry):
            j0 = 4 * t
            slot(j0, 0, 0)
            slot(j0 + 1, 1, 1)
            slot(j0 + 2, 2, 0)
            slot(j0 + 3, 3, 1)
            return carry

        lax.fori_loop(0, k // 4, step, 0)
        load_wait(k, 0)
        load_wait(k + 1, 1)
        idx_wait(k, 0)
        idx_wait(k + 1, 1)
        idx_wait(k + 2, 2)
        idx_wait(k + 3, 3)
        plsc.subcore_barrier()
        pltpu.sync_copy(acc.at[pl.ds(r0, rpt)], out_h.at[cid, pl.ds(r0, rpt)])

    kern = pl.kernel(
        body,
        out_type=[jax.ShapeDtypeStruct((NC, n_rows, 128), F32)],
        mesh=_sc_mesh(),
        scratch_types=[
            [pltpu.VMEM((CHUNK,), jnp.int32) for _ in range(4)],
            [pltpu.VMEM((CHUNK, 128), F32) for _ in range(2)],
            pltpu.VMEM_SHARED((n_rows, 128), F32),
            [pltpu.SemaphoreType.DMA for _ in range(4)],
            [pltpu.SemaphoreType.DMA for _ in range(2)],
        ],
    )
    return kern(ea4, dst4, zeros128)[0]


TC_BLK = 1000


def _tc_layer1(sa, ea, x, wl, we, wr, b, n):
    def body(sa_ref, ea_ref, x_ref, wl_ref, we_ref, wr_ref, b_ref, out_ref):
        s = sa_ref[0] + sa_ref[1]
        e = ea_ref[0] + ea_ref[1]
        inv = 1.0 / jnp.maximum(e[:, 16:17], 1.0)
        h = (
            jnp.dot(s * inv, wl_ref[...], preferred_element_type=F32,
                    precision=lax.Precision.HIGHEST)
            + jnp.dot(e[:, :16] * inv, we_ref[...], preferred_element_type=F32,
                      precision=lax.Precision.HIGHEST)
            + jnp.dot(x_ref[...], wr_ref[...], preferred_element_type=F32,
                      precision=lax.Precision.HIGHEST)
            + b_ref[...]
        )
        out_ref[...] = jnp.maximum(h, 0.0)

    full = lambda *shape: pl.BlockSpec(shape, lambda i: (0,) * len(shape))
    return pl.pallas_call(
        body,
        grid=(n // TC_BLK,),
        in_specs=[
            pl.BlockSpec((2, TC_BLK, 128), lambda i: (0, i, 0)),
            pl.BlockSpec((2, TC_BLK, 128), lambda i: (0, i, 0)),
            pl.BlockSpec((TC_BLK, 128), lambda i: (i, 0)),
            full(128, 128), full(16, 128), full(128, 128), full(128),
        ],
        out_specs=pl.BlockSpec((TC_BLK, 128), lambda i: (i, 0)),
        out_shape=jax.ShapeDtypeStruct((n, 128), F32),
    )(sa, ea, x, wl, we, wr, b)


def _tc_layer2_attn(sb, ea, h1, wl, we, wr, b, w1, b1, w2, n):
    def body(sb_ref, ea_ref, h1_ref, wl_ref, we_ref, wr_ref, b_ref,
             w1_ref, b1_ref, w2_ref, out_ref):
        s = sb_ref[0] + sb_ref[1]
        e = ea_ref[0] + ea_ref[1]
        inv = 1.0 / jnp.maximum(e[:, 16:17], 1.0)
        h2 = (
            jnp.dot(s * inv, wl_ref[...], preferred_element_type=F32,
                    precision=lax.Precision.HIGHEST)
            + jnp.dot(e[:, :16] * inv, we_ref[...], preferred_element_type=F32,
                      precision=lax.Precision.HIGHEST)
            + jnp.dot(h1_ref[...], wr_ref[...], preferred_element_type=F32,
                      precision=lax.Precision.HIGHEST)
            + b_ref[...]
        )
        h2a = h2[:, :64]
        h2b = h2[:, 64:]
        ua = jnp.tanh(jnp.dot(h2a, w1_ref[...], preferred_element_type=F32,
                              precision=lax.Precision.HIGHEST) + b1_ref[...])
        ub = jnp.tanh(jnp.dot(h2b, w1_ref[...], preferred_element_type=F32,
                              precision=lax.Precision.HIGHEST) + b1_ref[...])
        sa_s = jnp.dot(ua, w2_ref[...], preferred_element_type=F32,
                       precision=lax.Precision.HIGHEST)
        sb_s = jnp.dot(ub, w2_ref[...], preferred_element_type=F32,
                       precision=lax.Precision.HIGHEST)
        m = jnp.maximum(sa_s, sb_s)
        ea_w = jnp.exp(sa_s - m)
        eb_w = jnp.exp(sb_s - m)
        out_ref[...] = (ea_w * h2a + eb_w * h2b) / (ea_w + eb_w)

    full = lambda *shape: pl.BlockSpec(shape, lambda i: (0,) * len(shape))
    return pl.pallas_call(
        body,
        grid=(n // TC_BLK,),
        in_specs=[
            pl.BlockSpec((2, TC_BLK, 128), lambda i: (0, i, 0)),
            pl.BlockSpec((2, TC_BLK, 128), lambda i: (0, i, 0)),
            pl.BlockSpec((TC_BLK, 128), lambda i: (i, 0)),
            full(128, 128), full(16, 128), full(128, 128), full(128),
            full(64, 64), full(64), full(64, 1),
        ],
        out_specs=pl.BlockSpec((TC_BLK, 64), lambda i: (i, 0)),
        out_shape=jax.ShapeDtypeStruct((n, 64), F32),
    )(sb, ea, h1, wl, we, wr, b, w1, b1, w2)


def _block_diag(a, b):
    da0, da1 = a.shape
    db0, db1 = b.shape
    out = jnp.zeros((da0 + db0, da1 + db1), F32)
    out = out.at[:da0, :da1].set(a)
    out = out.at[da0:, da1:].set(b)
    return out


def kernel(x, edge_index, edge_attr, params):
    n, in_ch = x.shape
    e = edge_index.shape[1]
    hid = 64
    ech = edge_attr.shape[1]

    quantum = 4 * NW * CHUNK
    ep = ((e + quantum - 1) // quantum) * quantum
    n_rows = ((n + 1 + NS * 8 - 1) // (NS * 8)) * (NS * 8)

    k = ep // (NW * CHUNK)
    src4 = jnp.concatenate([
        jnp.zeros((ep,), jnp.int32).at[:e].set(edge_index[0])
        .reshape(NW, k, CHUNK),
        jnp.zeros((NW, 4, CHUNK), jnp.int32),
    ], axis=1)
    dst4 = jnp.concatenate([
        jnp.full((ep,), n, jnp.int32).at[:e].set(edge_index[1])
        .reshape(NW, k, CHUNK),
        jnp.full((NW, 4, CHUNK), n, jnp.int32),
    ], axis=1)
    ea128 = jnp.zeros((ep, 128), F32)
    ea128 = ea128.at[:e, :ech].set(edge_attr)
    ea128 = ea128.at[:e, 16].set(1.0)
    ea4 = jnp.concatenate([
        ea128.reshape(NW, k, CHUNK, 128),
        jnp.zeros((NW, 4, CHUNK, 128), F32),
    ], axis=1)

    c0, c1 = params["convs"][0], params["convs"][1]
    wl1 = _block_diag(c0[0]["Wl"][:64], c1[0]["Wl"][:64])
    we1 = jnp.concatenate([c0[0]["Wl"][64:], c1[0]["Wl"][64:]], axis=1)
    wr1 = _block_diag(c0[0]["Wr"], c1[0]["Wr"])
    b1v = jnp.concatenate([c0[0]["bl"] + c0[0]["br"],
                           c1[0]["bl"] + c1[0]["br"]])
    wl2 = _block_diag(c0[1]["Wl"][:64], c1[1]["Wl"][:64])
    we2 = jnp.concatenate([c0[1]["Wl"][64:], c1[1]["Wl"][64:]], axis=1)
    wr2 = _block_diag(c0[1]["Wr"], c1[1]["Wr"])
    b2v = jnp.concatenate([c0[1]["bl"] + c0[1]["br"],
                           c1[1]["bl"] + c1[1]["br"]])
    attn = params["attn"]

    eagg = _sc_edge_segsum(ea4, dst4, n_rows)
    sa = _sc_segsum(x, src4, dst4, n_rows)

    h1 = _tc_layer1(sa, eagg, x, wl1, we1, wr1, b1v, n)

    sb = _sc_segsum(h1, src4, dst4, n_rows)

    return _tc_layer2_attn(sb, eagg, h1, wl2, we2, wr2, b2v,
                           attn["W1"], attn["b1"], attn["W2"], n)


# speedup vs baseline: 1.3329x; 1.1573x over previous
import functools

import jax
import jax.numpy as jnp
from jax import lax
from jax.experimental import pallas as pl
from jax.experimental.pallas import tpu as pltpu
from jax.experimental.pallas import tpu_sc as plsc

NC = 2
NS = 16
NW = NC * NS
CHUNK = 128

F32 = jnp.float32


def _sc_mesh():
    return plsc.VectorSubcoreMesh(core_axis_name="c", subcore_axis_name="s",
                                  num_cores=NC, num_subcores=NS)


def _sc_segsum(feat, src2, dst2, n_rows, k):
    assert k % 4 == 0 and n_rows % (NS * 8) == 0
    rpt = n_rows // NS
    zeros128 = jnp.zeros((n_rows, 128), F32)

    def body(feat_h, s_h, d_h, z128_h, out_h, sidx, didx, rows, acc, isem,
             gsem):
        cid = lax.axis_index("c")
        sid = lax.axis_index("s")
        wid = sid * NC + cid
        r0 = sid * rpt
        c0 = wid * k

        pltpu.sync_copy(z128_h.at[pl.ds(r0, rpt)], acc.at[pl.ds(r0, rpt)])
        plsc.subcore_barrier()

        def idx_load(j, p4):
            jc = c0 + jnp.minimum(j, k - 1)
            pltpu.async_copy(s_h.at[jc], sidx[p4], isem[p4])
            pltpu.async_copy(d_h.at[jc], didx[p4], isem[p4])

        def idx_wait(j, p4):
            jc = c0 + jnp.minimum(j, k - 1)
            pltpu.make_async_copy(s_h.at[jc], sidx[p4], isem[p4]).wait()
            pltpu.make_async_copy(d_h.at[jc], didx[p4], isem[p4]).wait()

        for p in range(4):
            idx_load(p, p)

        def slot(j, p4):
            idx_wait(j, p4)
            pltpu.async_copy(feat_h.at[sidx[p4]], rows, gsem).wait()
            pltpu.sync_copy(rows, acc.at[didx[p4]], add=True)
            idx_load(j + 4, p4)

        def step(t, carry):
            j0 = 4 * t
            for p in range(4):
                slot(j0 + p, p)
            return carry

        lax.fori_loop(0, k // 4, step, 0)
        for p in range(4):
            idx_wait(k + p, p)
        plsc.subcore_barrier()

        pltpu.sync_copy(acc.at[pl.ds(r0, rpt)], out_h.at[cid, pl.ds(r0, rpt)])

    kern = pl.kernel(
        body,
        out_type=[jax.ShapeDtypeStruct((NC, n_rows, 128), F32)],
        mesh=_sc_mesh(),
        scratch_types=[
            [pltpu.VMEM((CHUNK,), jnp.int32) for _ in range(4)],
            [pltpu.VMEM((CHUNK,), jnp.int32) for _ in range(4)],
            pltpu.VMEM((CHUNK, 128), F32),
            pltpu.VMEM_SHARED((n_rows, 128), F32),
            [pltpu.SemaphoreType.DMA for _ in range(4)],
            pltpu.SemaphoreType.DMA,
        ],
    )
    return kern(feat, src2, dst2, zeros128)[0]


def _sc_edge_segsum(ea3, dst2, n_rows, k):
    assert k % 4 == 0
    rpt = n_rows // NS
    zeros128 = jnp.zeros((n_rows, 128), F32)

    def body(ea_h, d_h, z128_h, out_h, didx, eat, acc, isem, lsem):
        cid = lax.axis_index("c")
        sid = lax.axis_index("s")
        wid = sid * NC + cid
        r0 = sid * rpt
        c0 = wid * k
        pltpu.sync_copy(z128_h.at[pl.ds(r0, rpt)], acc.at[pl.ds(r0, rpt)])
        plsc.subcore_barrier()

        def idx_load(j, p4):
            pltpu.async_copy(d_h.at[c0 + jnp.minimum(j, k - 1)], didx[p4],
                             isem[p4])

        def idx_wait(j, p4):
            pltpu.make_async_copy(d_h.at[c0 + jnp.minimum(j, k - 1)],
                                  didx[p4], isem[p4]).wait()

        def load(j, p2):
            pltpu.async_copy(ea_h.at[c0 + jnp.minimum(j, k - 1)], eat[p2],
                             lsem[p2])

        def load_wait(j, p2):
            pltpu.make_async_copy(ea_h.at[c0 + jnp.minimum(j, k - 1)],
                                  eat[p2], lsem[p2]).wait()

        def scat(p4, p2):
            pltpu.sync_copy(eat[p2], acc.at[didx[p4]], add=True)

        for p in range(4):
            idx_load(p, p)
        load(0, 0)
        load(1, 1)

        def slot(j, p4, p2):
            load_wait(j, p2)
            idx_wait(j, p4)
            scat(p4, p2)
            load(j + 2, p2)
            idx_load(j + 4, p4)

        def step(t, carry):
            j0 = 4 * t
            slot(j0, 0, 0)
            slot(j0 + 1, 1, 1)
            slot(j0 + 2, 2, 0)
            slot(j0 + 3, 3, 1)
            return carry

        lax.fori_loop(0, k // 4, step, 0)
        load_wait(k, 0)
        load_wait(k + 1, 1)
        for p in range(4):
            idx_wait(k + p, p)
        plsc.subcore_barrier()
        pltpu.sync_copy(acc.at[pl.ds(r0, rpt)], out_h.at[cid, pl.ds(r0, rpt)])

    kern = pl.kernel(
        body,
        out_type=[jax.ShapeDtypeStruct((NC, n_rows, 128), F32)],
        mesh=_sc_mesh(),
        scratch_types=[
            [pltpu.VMEM((CHUNK,), jnp.int32) for _ in range(4)],
            [pltpu.VMEM((CHUNK, 128), F32) for _ in range(2)],
            pltpu.VMEM_SHARED((n_rows, 128), F32),
            [pltpu.SemaphoreType.DMA for _ in range(4)],
            [pltpu.SemaphoreType.DMA for _ in range(2)],
        ],
    )
    return kern(ea3, dst2, zeros128)[0]


TC_BLK = 1000


def _tc_layer1(sa, ea, x, wl, we, wr, b, n):
    def body(sa_ref, ea_ref, x_ref, wl_ref, we_ref, wr_ref, b_ref, out_ref):
        s = sa_ref[0] + sa_ref[1]
        e = ea_ref[0] + ea_ref[1]
        inv = 1.0 / jnp.maximum(e[:, 16:17], 1.0)
        h = (
            jnp.dot(s * inv, wl_ref[...], preferred_element_type=F32,
                    precision=lax.Precision.HIGHEST)
            + jnp.dot(e[:, :16] * inv, we_ref[...], preferred_element_type=F32,
                      precision=lax.Precision.HIGHEST)
            + jnp.dot(x_ref[...], wr_ref[...], preferred_element_type=F32,
                      precision=lax.Precision.HIGHEST)
            + b_ref[...]
        )
        out_ref[...] = jnp.maximum(h, 0.0)

    full = lambda *shape: pl.BlockSpec(shape, lambda i: (0,) * len(shape))
    return pl.pallas_call(
        body,
        grid=(n // TC_BLK,),
        in_specs=[
            pl.BlockSpec((2, TC_BLK, 128), lambda i: (0, i, 0)),
            pl.BlockSpec((2, TC_BLK, 128), lambda i: (0, i, 0)),
            pl.BlockSpec((TC_BLK, 128), lambda i: (i, 0)),
            full(128, 128), full(16, 128), full(128, 128), full(128),
        ],
        out_specs=pl.BlockSpec((TC_BLK, 128), lambda i: (i, 0)),
        out_shape=jax.ShapeDtypeStruct((n, 128), F32),
    )(sa, ea, x, wl, we, wr, b)


def _tc_layer2_attn(sb, ea, h1, wl, we, wr, b, w1, b1, w2, n):
    def body(sb_ref, ea_ref, h1_ref, wl_ref, we_ref, wr_ref, b_ref,
             w1_ref, b1_ref, w2_ref, out_ref):
        s = sb_ref[0] + sb_ref[1]
        e = ea_ref[0] + ea_ref[1]
        inv = 1.0 / jnp.maximum(e[:, 16:17], 1.0)
        h2 = (
            jnp.dot(s * inv, wl_ref[...], preferred_element_type=F32,
                    precision=lax.Precision.HIGHEST)
            + jnp.dot(e[:, :16] * inv, we_ref[...], preferred_element_type=F32,
                      precision=lax.Precision.HIGHEST)
            + jnp.dot(h1_ref[...], wr_ref[...], preferred_element_type=F32,
                      precision=lax.Precision.HIGHEST)
            + b_ref[...]
        )
        h2a = h2[:, :64]
        h2b = h2[:, 64:]
        ua = jnp.tanh(jnp.dot(h2a, w1_ref[...], preferred_element_type=F32,
                              precision=lax.Precision.HIGHEST) + b1_ref[...])
        ub = jnp.tanh(jnp.dot(h2b, w1_ref[...], preferred_element_type=F32,
                              precision=lax.Precision.HIGHEST) + b1_ref[...])
        sa_s = jnp.dot(ua, w2_ref[...], preferred_element_type=F32,
                       precision=lax.Precision.HIGHEST)
        sb_s = jnp.dot(ub, w2_ref[...], preferred_element_type=F32,
                       precision=lax.Precision.HIGHEST)
        m = jnp.maximum(sa_s, sb_s)
        ea_w = jnp.exp(sa_s - m)
        eb_w = jnp.exp(sb_s - m)
        out_ref[...] = (ea_w * h2a + eb_w * h2b) / (ea_w + eb_w)

    full = lambda *shape: pl.BlockSpec(shape, lambda i: (0,) * len(shape))
    return pl.pallas_call(
        body,
        grid=(n // TC_BLK,),
        in_specs=[
            pl.BlockSpec((2, TC_BLK, 128), lambda i: (0, i, 0)),
            pl.BlockSpec((2, TC_BLK, 128), lambda i: (0, i, 0)),
            pl.BlockSpec((TC_BLK, 128), lambda i: (i, 0)),
            full(128, 128), full(16, 128), full(128, 128), full(128),
            full(64, 64), full(64), full(64, 1),
        ],
        out_specs=pl.BlockSpec((TC_BLK, 64), lambda i: (i, 0)),
        out_shape=jax.ShapeDtypeStruct((n, 64), F32),
    )(sb, ea, h1, wl, we, wr, b, w1, b1, w2)


def _block_diag(a, b):
    da0, da1 = a.shape
    db0, db1 = b.shape
    out = jnp.zeros((da0 + db0, da1 + db1), F32)
    out = out.at[:da0, :da1].set(a)
    out = out.at[da0:, da1:].set(b)
    return out


def kernel(x, edge_index, edge_attr, params):
    n, in_ch = x.shape
    e = edge_index.shape[1]
    hid = 64
    ech = edge_attr.shape[1]

    quantum = 4 * NW * CHUNK
    ep = ((e + quantum - 1) // quantum) * quantum
    n_rows = ((n + 1 + NS * 8 - 1) // (NS * 8)) * (NS * 8)

    k = ep // (NW * CHUNK)
    src2 = (jnp.concatenate([edge_index[0],
                             jnp.zeros((ep - e,), jnp.int32)])
            .reshape(NW * k, CHUNK))
    dst2 = (jnp.concatenate([edge_index[1],
                             jnp.full((ep - e,), n, jnp.int32)])
            .reshape(NW * k, CHUNK))
    ea3 = jnp.concatenate([
        edge_attr,
        jnp.ones((e, 1), F32),
        jnp.zeros((e, 128 - ech - 1), F32),
    ], axis=1)
    ea3 = jnp.concatenate([ea3, jnp.zeros((ep - e, 128), F32)], axis=0)
    ea3 = ea3.reshape(NW * k, CHUNK, 128)

    c0, c1 = params["convs"][0], params["convs"][1]
    wl1 = _block_diag(c0[0]["Wl"][:64], c1[0]["Wl"][:64])
    we1 = jnp.concatenate([c0[0]["Wl"][64:], c1[0]["Wl"][64:]], axis=1)
    wr1 = _block_diag(c0[0]["Wr"], c1[0]["Wr"])
    b1v = jnp.concatenate([c0[0]["bl"] + c0[0]["br"],
                           c1[0]["bl"] + c1[0]["br"]])
    wl2 = _block_diag(c0[1]["Wl"][:64], c1[1]["Wl"][:64])
    we2 = jnp.concatenate([c0[1]["Wl"][64:], c1[1]["Wl"][64:]], axis=1)
    wr2 = _block_diag(c0[1]["Wr"], c1[1]["Wr"])
    b2v = jnp.concatenate([c0[1]["bl"] + c0[1]["br"],
                           c1[1]["bl"] + c1[1]["br"]])
    attn = params["attn"]

    eagg = _sc_edge_segsum(ea3, dst2, n_rows, k)
    sa = _sc_segsum(x, src2, dst2, n_rows, k)

    h1 = _tc_layer1(sa, eagg, x, wl1, we1, wr1, b1v, n)

    sb = _sc_segsum(h1, src2, dst2, n_rows, k)

    return _tc_layer2_attn(sb, eagg, h1, wl2, we2, wr2, b2v,
                           attn["W1"], attn["b1"], attn["W2"], n)
